# Initial kernel scaffold; baseline (speedup 1.0000x reference)
import jax
import jax.numpy as jnp
from jax.experimental import pallas as pl


def kernel(node_seq, edge_index, edge_attr, l0_Wih, l0_Whh, l0_bih, l0_bhh, l1_Wih, l1_Whh, l1_bih, l1_bhh, ne_W1, ne_b1, ne_W2, ne_b2, ee_W1, ee_b1, ee_W2, ee_b2, mp0_mW1, mp0_mb1, mp0_mW2, mp0_mb2, mp0_uW1, mp0_ub1, mp0_uW2, mp0_ub2, mp1_mW1, mp1_mb1, mp1_mW2, mp1_mb2, mp1_uW1, mp1_ub1, mp1_uW2, mp1_ub2, dec_W1, dec_b1, dec_W2, dec_b2):
    raise NotImplementedError("write your pallas kernel here")


# speedup vs baseline: 2.6942x; 2.6942x over previous
import functools

import jax
import jax.numpy as jnp
from jax import lax
from jax.experimental import pallas as pl
from jax.experimental.pallas import tpu as pltpu
from jax.experimental.pallas import tpu_sc as plsc

F32 = jnp.float32

N_NODES = 50000
N_EDGES = 800000
T_STEPS = 12
F_IN = 16
H = 64
HH = 32
K_IDX = 128
SB_ROWS = 2
N_SB = N_EDGES // (K_IDX * SB_ROWS)
NS = 16
ROWS_PER_W = N_NODES // NS


def _lstm_body(seq_ref, w0, u0, b0, w1, u1, b1, nw1, nb1, nw2, nb2, a0t,
               h_ref, ha_ref):
  x = seq_ref[...]
  bn = x.shape[0]
  h0 = jnp.zeros((bn, H), F32)
  c0 = jnp.zeros((bn, H), F32)
  h1 = jnp.zeros((bn, H), F32)
  c1 = jnp.zeros((bn, H), F32)
  W0 = w0[...]; U0 = u0[...]; B0 = b0[...]
  W1 = w1[...]; U1 = u1[...]; B1 = b1[...]
  for t in range(T_STEPS):
    xt = x[:, t * F_IN:(t + 1) * F_IN]
    g = (jnp.dot(xt, W0, preferred_element_type=F32)
         + jnp.dot(h0, U0, preferred_element_type=F32) + B0)
    ig = jax.nn.sigmoid(g[:, :H])
    fg = jax.nn.sigmoid(g[:, H:2 * H])
    gg = jnp.tanh(g[:, 2 * H:3 * H])
    og = jax.nn.sigmoid(g[:, 3 * H:])
    c0 = fg * c0 + ig * gg
    h0 = og * jnp.tanh(c0)
    g = (jnp.dot(h0, W1, preferred_element_type=F32)
         + jnp.dot(h1, U1, preferred_element_type=F32) + B1)
    ig = jax.nn.sigmoid(g[:, :H])
    fg = jax.nn.sigmoid(g[:, H:2 * H])
    gg = jnp.tanh(g[:, 2 * H:3 * H])
    og = jax.nn.sigmoid(g[:, 3 * H:])
    c1 = fg * c1 + ig * gg
    h1 = og * jnp.tanh(c1)
  h = jax.nn.relu(jnp.dot(h1, nw1[...], preferred_element_type=F32) + nb1[...])
  h = jax.nn.relu(jnp.dot(h, nw2[...], preferred_element_type=F32) + nb2[...])
  h_ref[...] = h
  ha = jnp.dot(h, a0t[...], preferred_element_type=F32)
  ha_ref[0, :, :] = ha[:, :HH]
  ha_ref[1, :, :] = ha[:, HH:]


def _lstm_encode(seq_flat, w0, u0, b0, w1, u1, b1, nw1, nb1, nw2, nb2, a0t):
  bn = 2000
  grid = (N_NODES // bn,)

  def wspec(w):
    r = w.ndim
    return pl.BlockSpec(w.shape, lambda i, _r=r: (0,) * _r)

  ws = [w0, u0, b0, w1, u1, b1, nw1, nb1, nw2, nb2, a0t]
  return pl.pallas_call(
      _lstm_body,
      grid=grid,
      in_specs=[pl.BlockSpec((bn, T_STEPS * F_IN), lambda i: (i, 0))]
      + [wspec(w) for w in ws],
      out_specs=[
          pl.BlockSpec((bn, H), lambda i: (i, 0)),
          pl.BlockSpec((2, bn, HH), lambda i: (0, i, 0)),
      ],
      out_shape=[
          jax.ShapeDtypeStruct((N_NODES, H), F32),
          jax.ShapeDtypeStruct((2, N_NODES, HH), F32),
      ],
  )(seq_flat, *ws)


def _edge_body(ea_ref, ew, eb, c0t, d0, c1t, d1, e0_ref, e1_ref):
  r = jax.nn.relu(
      jnp.dot(ea_ref[...], ew[...], preferred_element_type=F32) + eb[...])
  v0 = jnp.dot(r, c0t[...], preferred_element_type=F32) + d0[...]
  v1 = jnp.dot(r, c1t[...], preferred_element_type=F32) + d1[...]
  e0_ref[0, :, :] = v0[:, :HH]
  e0_ref[1, :, :] = v0[:, HH:]
  e1_ref[0, :, :] = v1[:, :HH]
  e1_ref[1, :, :] = v1[:, HH:]


def _edge_prep(ea_pad, ew, eb, c0t, d0, c1t, d1):
  be = 2000
  grid = (N_EDGES // be,)

  def wspec(w):
    r = w.ndim
    return pl.BlockSpec(w.shape, lambda i, _r=r: (0,) * _r)

  ws = [ew, eb, c0t, d0, c1t, d1]
  return pl.pallas_call(
      _edge_body,
      grid=grid,
      in_specs=[pl.BlockSpec((be, 8), lambda i: (i, 0))]
      + [wspec(w) for w in ws],
      out_specs=[
          pl.BlockSpec((2, be, HH), lambda i: (0, i, 0)),
          pl.BlockSpec((2, be, HH), lambda i: (0, i, 0)),
      ],
      out_shape=[
          jax.ShapeDtypeStruct((2, N_EDGES, HH), F32),
          jax.ShapeDtypeStruct((2, N_EDGES, HH), F32),
      ],
  )(ea_pad, *ws)


def _sc_body(srcs, dsts, ha, eb3, zeros_f,
             out_feat,
             acc_f, src_v, dst_v, gidx_v, gath_v, eb_v, sem):
  c = lax.axis_index("c")
  s = lax.axis_index("s")

  pltpu.sync_copy(zeros_f, acc_f.at[pl.ds(s * ROWS_PER_W, ROWS_PER_W)])
  plsc.subcore_barrier()

  coff = c * N_NODES
  ebase = c * (N_SB * SB_ROWS)
  n_mine = (N_SB - s + NS - 1) // NS

  def superblock(i, carry):
    sb = s + i * NS
    row0 = sb * SB_ROWS
    pltpu.sync_copy(srcs.at[pl.ds(row0, SB_ROWS)], src_v)
    pltpu.sync_copy(dsts.at[pl.ds(row0, SB_ROWS)], dst_v)
    for r in range(SB_ROWS):
      for k in range(K_IDX // 16):
        gidx_v[r, pl.ds(k * 16, 16)] = src_v[r, pl.ds(k * 16, 16)] + coff
    descs = [
        pltpu.async_copy(ha.at[gidx_v.at[r]], gath_v.at[r], sem)
        for r in range(SB_ROWS)
    ]
    pltpu.sync_copy(eb3.at[pl.ds(ebase + row0, SB_ROWS)], eb_v)
    for d in descs:
      d.wait()
    for r in range(SB_ROWS):
      def rowgrp(j, carry2, _r=r):
        for jj in range(4):
          row = j * 4 + jj
          for w in range(HH // 16):
            sl = pl.ds(w * 16, 16)
            eb_v[_r, row, sl] = jnp.maximum(
                gath_v[_r, row, sl] + eb_v[_r, row, sl], 0.0)
        return carry2
      lax.fori_loop(0, K_IDX // 4, rowgrp, 0)
    for r in range(SB_ROWS):
      pltpu.sync_copy(eb_v.at[r], acc_f.at[dst_v.at[r]], add=True)
    return carry

  lax.fori_loop(0, n_mine, superblock, 0)
  plsc.subcore_barrier()

  pltpu.sync_copy(acc_f.at[pl.ds(s * ROWS_PER_W, ROWS_PER_W)],
                  out_feat.at[pl.ds(coff + s * ROWS_PER_W, ROWS_PER_W)])


@functools.cache
def _get_sc_mp():
  return functools.partial(
      pl.kernel,
      out_type=jax.ShapeDtypeStruct((2 * N_NODES, HH), F32),
      mesh=plsc.VectorSubcoreMesh(core_axis_name="c", subcore_axis_name="s"),
      compiler_params=pltpu.CompilerParams(use_tc_tiling_on_sc=False),
      scratch_types=[
          pltpu.VMEM_SHARED((N_NODES, HH), F32),
          pltpu.VMEM((SB_ROWS, K_IDX), jnp.int32),
          pltpu.VMEM((SB_ROWS, K_IDX), jnp.int32),
          pltpu.VMEM((SB_ROWS, K_IDX), jnp.int32),
          pltpu.VMEM((SB_ROWS, K_IDX, HH), F32),
          pltpu.VMEM((SB_ROWS, K_IDX, HH), F32),
          pltpu.SemaphoreType.DMA,
      ],
  )(_sc_body)


def _sc_mp(*args):
  return _get_sc_mp()(*args)


CNT_SB = 5
CNT_ROWS_PER_CORE = (N_EDGES // K_IDX) // 2
CNT_NSB = CNT_ROWS_PER_CORE // CNT_SB


def _sc_cnt_body(dsts, zeros_c, ones_h, out_cnt,
                 acc_c, dst_v, ones_v, sem):
  c = lax.axis_index("c")
  s = lax.axis_index("s")

  pltpu.sync_copy(zeros_c, acc_c.at[pl.ds(s * ROWS_PER_W, ROWS_PER_W)])
  pltpu.sync_copy(ones_h, ones_v)
  plsc.subcore_barrier()

  rbase = c * CNT_ROWS_PER_CORE
  n_mine = (CNT_NSB - s + NS - 1) // NS

  def superblock(i, carry):
    row0 = rbase + (s + i * NS) * CNT_SB
    pltpu.sync_copy(dsts.at[pl.ds(row0, CNT_SB)], dst_v)
    for r in range(CNT_SB):
      pltpu.sync_copy(ones_v, acc_c.at[dst_v.at[r]], add=True)
    return carry

  lax.fori_loop(0, n_mine, superblock, 0)
  plsc.subcore_barrier()

  pltpu.sync_copy(acc_c.at[pl.ds(s * ROWS_PER_W, ROWS_PER_W)],
                  out_cnt.at[pl.ds(c * N_NODES + s * ROWS_PER_W, ROWS_PER_W)])


@functools.cache
def _get_sc_cnt():
  return functools.partial(
      pl.kernel,
      out_type=jax.ShapeDtypeStruct((2 * N_NODES, 8), F32),
      mesh=plsc.VectorSubcoreMesh(core_axis_name="c", subcore_axis_name="s"),
      compiler_params=pltpu.CompilerParams(use_tc_tiling_on_sc=False),
      scratch_types=[
          pltpu.VMEM_SHARED((N_NODES, 8), F32),
          pltpu.VMEM((CNT_SB, K_IDX), jnp.int32),
          pltpu.VMEM((K_IDX, 8), F32),
          pltpu.SemaphoreType.DMA,
      ],
  )(_sc_cnt_body)


def _sc_cnt(*args):
  return _get_sc_cnt()(*args)


def _update_body(h_ref, sa_ref, sb_ref, ca_ref, cb_ref, m2t, mb2, u1at, u1bt,
                 ub1, u2t, ub2, a1t, h_out, ha_out):
  h = h_ref[...]
  cnt = (ca_ref[...] + cb_ref[...])[:, 0:1]
  S = jnp.concatenate([sa_ref[...], sb_ref[...]], axis=1)
  inv = 1.0 / jnp.maximum(cnt, 1.0)
  ind = (cnt > 0.0).astype(F32)
  aggr = jnp.dot(S * inv, m2t[...], preferred_element_type=F32) + mb2[...] * ind
  u = jax.nn.relu(
      jnp.dot(h, u1at[...], preferred_element_type=F32)
      + jnp.dot(aggr, u1bt[...], preferred_element_type=F32) + ub1[...])
  u = jnp.dot(u, u2t[...], preferred_element_type=F32) + ub2[...]
  hn = h + u
  h_out[...] = hn
  ha = jnp.dot(hn, a1t[...], preferred_element_type=F32)
  ha_out[0, :, :] = ha[:, :HH]
  ha_out[1, :, :] = ha[:, HH:]


def _update_dec_body(h_ref, sa_ref, sb_ref, ca_ref, cb_ref, m2t, mb2, u1at,
                     u1bt, ub1, u2t, ub2, dw1t, db1, dw2t, db2, out_ref):
  h = h_ref[...]
  cnt = (ca_ref[...] + cb_ref[...])[:, 0:1]
  S = jnp.concatenate([sa_ref[...], sb_ref[...]], axis=1)
  inv = 1.0 / jnp.maximum(cnt, 1.0)
  ind = (cnt > 0.0).astype(F32)
  aggr = jnp.dot(S * inv, m2t[...], preferred_element_type=F32) + mb2[...] * ind
  u = jax.nn.relu(
      jnp.dot(h, u1at[...], preferred_element_type=F32)
      + jnp.dot(aggr, u1bt[...], preferred_element_type=F32) + ub1[...])
  u = jnp.dot(u, u2t[...], preferred_element_type=F32) + ub2[...]
  hn = h + u
  d = jax.nn.relu(jnp.dot(hn, dw1t[...], preferred_element_type=F32) + db1[...])
  d = jnp.dot(d, dw2t[...], preferred_element_type=F32) + db2[...]
  out_ref[...] = jax.nn.sigmoid(d)


def _update(h, out_feat, out_cnt, ws, body, out_specs, out_shape):
  bn = 2000
  nblk = N_NODES // bn
  grid = (nblk,)

  def wspec(w):
    r = w.ndim
    return pl.BlockSpec(w.shape, lambda i, _r=r: (0,) * _r)

  return pl.pallas_call(
      body,
      grid=grid,
      in_specs=[
          pl.BlockSpec((bn, H), lambda i: (i, 0)),
          pl.BlockSpec((bn, HH), lambda i: (i, 0)),
          pl.BlockSpec((bn, HH), lambda i, _n=nblk: (i + _n, 0)),
          pl.BlockSpec((bn, 8), lambda i: (i, 0)),
          pl.BlockSpec((bn, 8), lambda i, _n=nblk: (i + _n, 0)),
      ] + [wspec(w) for w in ws],
      out_specs=out_specs,
      out_shape=out_shape,
  )(h, out_feat, out_feat, out_cnt, out_cnt, *ws)


def kernel(node_seq, edge_index, edge_attr,
           l0_Wih, l0_Whh, l0_bih, l0_bhh,
           l1_Wih, l1_Whh, l1_bih, l1_bhh,
           ne_W1, ne_b1, ne_W2, ne_b2,
           ee_W1, ee_b1, ee_W2, ee_b2,
           mp0_mW1, mp0_mb1, mp0_mW2, mp0_mb2,
           mp0_uW1, mp0_ub1, mp0_uW2, mp0_ub2,
           mp1_mW1, mp1_mb1, mp1_mW2, mp1_mb2,
           mp1_uW1, mp1_ub1, mp1_uW2, mp1_ub2,
           dec_W1, dec_b1, dec_W2, dec_b2):
  r2 = lambda v: v.reshape(1, -1)

  seq_flat = node_seq.reshape(N_NODES, T_STEPS * F_IN)
  a0 = mp0_mW1[:, :H]
  h, ha0 = _lstm_encode(
      seq_flat, l0_Wih.T, l0_Whh.T, r2(l0_bih + l0_bhh),
      l1_Wih.T, l1_Whh.T, r2(l1_bih + l1_bhh),
      ne_W1.T, r2(ne_b1), ne_W2.T, r2(ne_b2), a0.T)

  ea_pad = jnp.pad(edge_attr, ((0, 0), (0, 4)))
  ew = jnp.pad(ee_W1.T, ((0, 4), (0, 0)))
  b0m = mp0_mW1[:, H:]
  b1m = mp1_mW1[:, H:]
  c0 = b0m @ ee_W2
  d0 = b0m @ ee_b2 + mp0_mb1
  c1 = b1m @ ee_W2
  d1 = b1m @ ee_b2 + mp1_mb1
  eb0, eb1 = _edge_prep(ea_pad, ew, r2(ee_b1), c0.T, r2(d0), c1.T, r2(d1))

  srcs = edge_index[0].reshape(N_EDGES // K_IDX, K_IDX)
  dsts = edge_index[1].reshape(N_EDGES // K_IDX, K_IDX)
  zeros_f = jnp.zeros((ROWS_PER_W, HH), F32)
  zeros_c = jnp.zeros((ROWS_PER_W, 8), F32)
  ones_h = jnp.ones((K_IDX, 8), F32)

  cntp = _sc_cnt(dsts, zeros_c, ones_h)

  ha_flat = ha0.reshape(2 * N_NODES, HH)
  eb3 = eb0.reshape(2 * (N_EDGES // K_IDX), K_IDX, HH)
  s0 = _sc_mp(srcs, dsts, ha_flat, eb3, zeros_f)
  ws0 = [mp0_mW2.T, r2(mp0_mb2), mp0_uW1[:, :H].T, mp0_uW1[:, H:].T,
         r2(mp0_ub1), mp0_uW2.T, r2(mp0_ub2), mp1_mW1[:, :H].T]
  bn = 2000
  h, ha1 = _update(
      h, s0, cntp, ws0, _update_body,
      out_specs=[
          pl.BlockSpec((bn, H), lambda i: (i, 0)),
          pl.BlockSpec((2, bn, HH), lambda i: (0, i, 0)),
      ],
      out_shape=[
          jax.ShapeDtypeStruct((N_NODES, H), F32),
          jax.ShapeDtypeStruct((2, N_NODES, HH), F32),
      ])

  ha_flat = ha1.reshape(2 * N_NODES, HH)
  eb3 = eb1.reshape(2 * (N_EDGES // K_IDX), K_IDX, HH)
  s1 = _sc_mp(srcs, dsts, ha_flat, eb3, zeros_f)
  ws1 = [mp1_mW2.T, r2(mp1_mb2), mp1_uW1[:, :H].T, mp1_uW1[:, H:].T,
         r2(mp1_ub1), mp1_uW2.T, r2(mp1_ub2),
         dec_W1.T, r2(dec_b1), dec_W2.T, r2(dec_b2)]
  out = _update(
      h, s1, cntp, ws1, _update_dec_body,
      out_specs=[pl.BlockSpec((bn, 1), lambda i: (i, 0))],
      out_shape=[jax.ShapeDtypeStruct((N_NODES, 1), F32)])
  return out[0].reshape(N_NODES)


# speedup vs baseline: 3.9662x; 1.4721x over previous
import functools

import jax
import jax.numpy as jnp
from jax import lax
from jax.experimental import pallas as pl
from jax.experimental.pallas import tpu as pltpu
from jax.experimental.pallas import tpu_sc as plsc

F32 = jnp.float32

N_NODES = 50000
N_EDGES = 800000
T_STEPS = 12
F_IN = 16
H = 64
HH = 32
K_IDX = 128
SB_ROWS = 2
N_SB = N_EDGES // (K_IDX * SB_ROWS)
NS = 16
ROWS_PER_W = N_NODES // NS


def _lstm_body(seq_ref, w0, u0, b0, w1, u1, b1, nw1, nb1, nw2, nb2, a0t,
               h_ref, ha_ref):
  x = seq_ref[...]
  bn = x.shape[0]
  h0 = jnp.zeros((bn, H), F32)
  c0 = jnp.zeros((bn, H), F32)
  h1 = jnp.zeros((bn, H), F32)
  c1 = jnp.zeros((bn, H), F32)
  W0 = w0[...]; U0 = u0[...]; B0 = b0[...]
  W1 = w1[...]; U1 = u1[...]; B1 = b1[...]
  for t in range(T_STEPS):
    xt = x[:, t * F_IN:(t + 1) * F_IN]
    g = (jnp.dot(xt, W0, preferred_element_type=F32)
         + jnp.dot(h0, U0, preferred_element_type=F32) + B0)
    ig = jax.nn.sigmoid(g[:, :H])
    fg = jax.nn.sigmoid(g[:, H:2 * H])
    gg = jnp.tanh(g[:, 2 * H:3 * H])
    og = jax.nn.sigmoid(g[:, 3 * H:])
    c0 = fg * c0 + ig * gg
    h0 = og * jnp.tanh(c0)
    g = (jnp.dot(h0, W1, preferred_element_type=F32)
         + jnp.dot(h1, U1, preferred_element_type=F32) + B1)
    ig = jax.nn.sigmoid(g[:, :H])
    fg = jax.nn.sigmoid(g[:, H:2 * H])
    gg = jnp.tanh(g[:, 2 * H:3 * H])
    og = jax.nn.sigmoid(g[:, 3 * H:])
    c1 = fg * c1 + ig * gg
    h1 = og * jnp.tanh(c1)
  h = jax.nn.relu(jnp.dot(h1, nw1[...], preferred_element_type=F32) + nb1[...])
  h = jax.nn.relu(jnp.dot(h, nw2[...], preferred_element_type=F32) + nb2[...])
  h_ref[...] = h
  ha = jnp.dot(h, a0t[...], preferred_element_type=F32)
  ha_ref[0, :, :] = ha[:, :HH]
  ha_ref[1, :, :] = ha[:, HH:]


def _lstm_encode(seq_flat, w0, u0, b0, w1, u1, b1, nw1, nb1, nw2, nb2, a0t):
  bn = 2000
  grid = (N_NODES // bn,)

  def wspec(w):
    r = w.ndim
    return pl.BlockSpec(w.shape, lambda i, _r=r: (0,) * _r)

  ws = [w0, u0, b0, w1, u1, b1, nw1, nb1, nw2, nb2, a0t]
  return pl.pallas_call(
      _lstm_body,
      grid=grid,
      in_specs=[pl.BlockSpec((bn, T_STEPS * F_IN), lambda i: (i, 0))]
      + [wspec(w) for w in ws],
      out_specs=[
          pl.BlockSpec((bn, H), lambda i: (i, 0)),
          pl.BlockSpec((2, bn, HH), lambda i: (0, i, 0)),
      ],
      out_shape=[
          jax.ShapeDtypeStruct((N_NODES, H), F32),
          jax.ShapeDtypeStruct((2, N_NODES, HH), F32),
      ],
  )(seq_flat, *ws)


def _edge_body(ea_ref, ew8, eb1c, c0t, d0, c1t, d1, out_ref):
  zt = jnp.dot(ew8[...], ea_ref[...], preferred_element_type=F32) + eb1c[...]
  rt = jax.nn.relu(zt)
  dn = (((0,), (0,)), ((), ()))
  v0 = lax.dot_general(rt, c0t[...], dn, preferred_element_type=F32) + d0[...]
  v1 = lax.dot_general(rt, c1t[...], dn, preferred_element_type=F32) + d1[...]
  out_ref[:, :H] = v0
  out_ref[:, H:] = v1


def _edge_prep(ea_t8, ew8, eb1c, c0t, d0, c1t, d1):
  be = 3200
  grid = (N_EDGES // be,)

  def wspec(w):
    r = w.ndim
    return pl.BlockSpec(w.shape, lambda i, _r=r: (0,) * _r)

  ws = [ew8, eb1c, c0t, d0, c1t, d1]
  return pl.pallas_call(
      _edge_body,
      grid=grid,
      in_specs=[pl.BlockSpec((8, be), lambda i: (0, i))]
      + [wspec(w) for w in ws],
      out_specs=pl.BlockSpec((be, 2 * H), lambda i: (i, 0)),
      out_shape=jax.ShapeDtypeStruct((N_EDGES, 2 * H), F32),
  )(ea_t8, *ws)


def _sc_body(lane0, srcs, dsts, ha, eb3, zeros_f,
             out_feat,
             acc_f, src_v, dst_v, gidx_v, gath_v, eb_v, sem):
  c = lax.axis_index("c")
  s = lax.axis_index("s")

  pltpu.sync_copy(zeros_f, acc_f.at[pl.ds(s * ROWS_PER_W, ROWS_PER_W)])
  plsc.subcore_barrier()

  coff = c * N_NODES
  eoff = lane0 + c * HH
  n_mine = (N_SB - s + NS - 1) // NS

  def superblock(i, carry):
    sb = s + i * NS
    row0 = sb * SB_ROWS
    pltpu.sync_copy(srcs.at[pl.ds(row0, SB_ROWS)], src_v)
    pltpu.sync_copy(dsts.at[pl.ds(row0, SB_ROWS)], dst_v)
    for r in range(SB_ROWS):
      for k in range(K_IDX // 16):
        gidx_v[r, pl.ds(k * 16, 16)] = src_v[r, pl.ds(k * 16, 16)] + coff
    descs = [
        pltpu.async_copy(ha.at[gidx_v.at[r]], gath_v.at[r], sem)
        for r in range(SB_ROWS)
    ]
    pltpu.sync_copy(
        eb3.at[pl.ds(row0, SB_ROWS), slice(None), pl.ds(eoff, HH)], eb_v)
    for d in descs:
      d.wait()
    for r in range(SB_ROWS):
      def rowgrp(j, carry2, _r=r):
        for jj in range(4):
          row = j * 4 + jj
          for w in range(HH // 16):
            sl = pl.ds(w * 16, 16)
            eb_v[_r, row, sl] = jnp.maximum(
                gath_v[_r, row, sl] + eb_v[_r, row, sl], 0.0)
        return carry2
      lax.fori_loop(0, K_IDX // 4, rowgrp, 0)
    for r in range(SB_ROWS):
      pltpu.sync_copy(eb_v.at[r], acc_f.at[dst_v.at[r]], add=True)
    return carry

  lax.fori_loop(0, n_mine, superblock, 0)
  plsc.subcore_barrier()

  pltpu.sync_copy(acc_f.at[pl.ds(s * ROWS_PER_W, ROWS_PER_W)],
                  out_feat.at[pl.ds(coff + s * ROWS_PER_W, ROWS_PER_W)])


@functools.cache
def _get_sc_mp(lane0):
  return functools.partial(
      pl.kernel,
      out_type=jax.ShapeDtypeStruct((2 * N_NODES, HH), F32),
      mesh=plsc.VectorSubcoreMesh(core_axis_name="c", subcore_axis_name="s"),
      compiler_params=pltpu.CompilerParams(use_tc_tiling_on_sc=False),
      scratch_types=[
          pltpu.VMEM_SHARED((N_NODES, HH), F32),
          pltpu.VMEM((SB_ROWS, K_IDX), jnp.int32),
          pltpu.VMEM((SB_ROWS, K_IDX), jnp.int32),
          pltpu.VMEM((SB_ROWS, K_IDX), jnp.int32),
          pltpu.VMEM((SB_ROWS, K_IDX, HH), F32),
          pltpu.VMEM((SB_ROWS, K_IDX, HH), F32),
          pltpu.SemaphoreType.DMA,
      ],
  )(functools.partial(_sc_body, lane0))


def _sc_mp(lane0, *args):
  return _get_sc_mp(lane0)(*args)


CNT_SB = 5
CNT_ROWS_PER_CORE = (N_EDGES // K_IDX) // 2
CNT_NSB = CNT_ROWS_PER_CORE // CNT_SB


def _sc_cnt_body(dsts, zeros_c, ones_h, out_cnt,
                 acc_c, dst_v, ones_v, sem):
  c = lax.axis_index("c")
  s = lax.axis_index("s")

  pltpu.sync_copy(zeros_c, acc_c.at[pl.ds(s * ROWS_PER_W, ROWS_PER_W)])
  pltpu.sync_copy(ones_h, ones_v)
  plsc.subcore_barrier()

  rbase = c * CNT_ROWS_PER_CORE
  n_mine = (CNT_NSB - s + NS - 1) // NS

  def superblock(i, carry):
    row0 = rbase + (s + i * NS) * CNT_SB
    pltpu.sync_copy(dsts.at[pl.ds(row0, CNT_SB)], dst_v)
    for r in range(CNT_SB):
      pltpu.sync_copy(ones_v, acc_c.at[dst_v.at[r]], add=True)
    return carry

  lax.fori_loop(0, n_mine, superblock, 0)
  plsc.subcore_barrier()

  pltpu.sync_copy(acc_c.at[pl.ds(s * ROWS_PER_W, ROWS_PER_W)],
                  out_cnt.at[pl.ds(c * N_NODES + s * ROWS_PER_W, ROWS_PER_W)])


@functools.cache
def _get_sc_cnt():
  return functools.partial(
      pl.kernel,
      out_type=jax.ShapeDtypeStruct((2 * N_NODES, 8), F32),
      mesh=plsc.VectorSubcoreMesh(core_axis_name="c", subcore_axis_name="s"),
      compiler_params=pltpu.CompilerParams(use_tc_tiling_on_sc=False),
      scratch_types=[
          pltpu.VMEM_SHARED((N_NODES, 8), F32),
          pltpu.VMEM((CNT_SB, K_IDX), jnp.int32),
          pltpu.VMEM((K_IDX, 8), F32),
          pltpu.SemaphoreType.DMA,
      ],
  )(_sc_cnt_body)


def _sc_cnt(*args):
  return _get_sc_cnt()(*args)


def _update_body(h_ref, sa_ref, sb_ref, ca_ref, cb_ref, m2t, mb2, u1at, u1bt,
                 ub1, u2t, ub2, a1t, h_out, ha_out):
  h = h_ref[...]
  cnt = (ca_ref[...] + cb_ref[...])[:, 0:1]
  S = jnp.concatenate([sa_ref[...], sb_ref[...]], axis=1)
  inv = 1.0 / jnp.maximum(cnt, 1.0)
  ind = (cnt > 0.0).astype(F32)
  aggr = jnp.dot(S * inv, m2t[...], preferred_element_type=F32) + mb2[...] * ind
  u = jax.nn.relu(
      jnp.dot(h, u1at[...], preferred_element_type=F32)
      + jnp.dot(aggr, u1bt[...], preferred_element_type=F32) + ub1[...])
  u = jnp.dot(u, u2t[...], preferred_element_type=F32) + ub2[...]
  hn = h + u
  h_out[...] = hn
  ha = jnp.dot(hn, a1t[...], preferred_element_type=F32)
  ha_out[0, :, :] = ha[:, :HH]
  ha_out[1, :, :] = ha[:, HH:]


def _update_dec_body(h_ref, sa_ref, sb_ref, ca_ref, cb_ref, m2t, mb2, u1at,
                     u1bt, ub1, u2t, ub2, dw1t, db1, dw2t, db2, out_ref):
  h = h_ref[...]
  cnt = (ca_ref[...] + cb_ref[...])[:, 0:1]
  S = jnp.concatenate([sa_ref[...], sb_ref[...]], axis=1)
  inv = 1.0 / jnp.maximum(cnt, 1.0)
  ind = (cnt > 0.0).astype(F32)
  aggr = jnp.dot(S * inv, m2t[...], preferred_element_type=F32) + mb2[...] * ind
  u = jax.nn.relu(
      jnp.dot(h, u1at[...], preferred_element_type=F32)
      + jnp.dot(aggr, u1bt[...], preferred_element_type=F32) + ub1[...])
  u = jnp.dot(u, u2t[...], preferred_element_type=F32) + ub2[...]
  hn = h + u
  d = jax.nn.relu(jnp.dot(hn, dw1t[...], preferred_element_type=F32) + db1[...])
  d = jnp.dot(d, dw2t[...], preferred_element_type=F32) + db2[...]
  out_ref[...] = jax.nn.sigmoid(d)


def _update(h, out_feat, out_cnt, ws, body, out_specs, out_shape):
  bn = 2000
  nblk = N_NODES // bn
  grid = (nblk,)

  def wspec(w):
    r = w.ndim
    return pl.BlockSpec(w.shape, lambda i, _r=r: (0,) * _r)

  return pl.pallas_call(
      body,
      grid=grid,
      in_specs=[
          pl.BlockSpec((bn, H), lambda i: (i, 0)),
          pl.BlockSpec((bn, HH), lambda i: (i, 0)),
          pl.BlockSpec((bn, HH), lambda i, _n=nblk: (i + _n, 0)),
          pl.BlockSpec((bn, 8), lambda i: (i, 0)),
          pl.BlockSpec((bn, 8), lambda i, _n=nblk: (i + _n, 0)),
      ] + [wspec(w) for w in ws],
      out_specs=out_specs,
      out_shape=out_shape,
  )(h, out_feat, out_feat, out_cnt, out_cnt, *ws)


def kernel(node_seq, edge_index, edge_attr,
           l0_Wih, l0_Whh, l0_bih, l0_bhh,
           l1_Wih, l1_Whh, l1_bih, l1_bhh,
           ne_W1, ne_b1, ne_W2, ne_b2,
           ee_W1, ee_b1, ee_W2, ee_b2,
           mp0_mW1, mp0_mb1, mp0_mW2, mp0_mb2,
           mp0_uW1, mp0_ub1, mp0_uW2, mp0_ub2,
           mp1_mW1, mp1_mb1, mp1_mW2, mp1_mb2,
           mp1_uW1, mp1_ub1, mp1_uW2, mp1_ub2,
           dec_W1, dec_b1, dec_W2, dec_b2):
  r2 = lambda v: v.reshape(1, -1)

  seq_flat = node_seq.reshape(N_NODES, T_STEPS * F_IN)
  a0 = mp0_mW1[:, :H]
  h, ha0 = _lstm_encode(
      seq_flat, l0_Wih.T, l0_Whh.T, r2(l0_bih + l0_bhh),
      l1_Wih.T, l1_Whh.T, r2(l1_bih + l1_bhh),
      ne_W1.T, r2(ne_b1), ne_W2.T, r2(ne_b2), a0.T)

  ea_t8 = jnp.pad(edge_attr.T, ((0, 4), (0, 0)))
  ew8 = jnp.pad(ee_W1, ((0, 0), (0, 4)))
  b0m = mp0_mW1[:, H:]
  b1m = mp1_mW1[:, H:]
  c0 = b0m @ ee_W2
  d0 = b0m @ ee_b2 + mp0_mb1
  c1 = b1m @ ee_W2
  d1 = b1m @ ee_b2 + mp1_mb1
  ebf = _edge_prep(ea_t8, ew8, ee_b1.reshape(-1, 1), c0.T, r2(d0), c1.T,
                   r2(d1))
  eb3 = ebf.reshape(N_EDGES // K_IDX, K_IDX, 2 * H)

  srcs = edge_index[0].reshape(N_EDGES // K_IDX, K_IDX)
  dsts = edge_index[1].reshape(N_EDGES // K_IDX, K_IDX)
  zeros_f = jnp.zeros((ROWS_PER_W, HH), F32)
  zeros_c = jnp.zeros((ROWS_PER_W, 8), F32)
  ones_h = jnp.ones((K_IDX, 8), F32)

  cntp = _sc_cnt(dsts, zeros_c, ones_h)

  ha_flat = ha0.reshape(2 * N_NODES, HH)
  s0 = _sc_mp(0, srcs, dsts, ha_flat, eb3, zeros_f)
  ws0 = [mp0_mW2.T, r2(mp0_mb2), mp0_uW1[:, :H].T, mp0_uW1[:, H:].T,
         r2(mp0_ub1), mp0_uW2.T, r2(mp0_ub2), mp1_mW1[:, :H].T]
  bn = 2000
  h, ha1 = _update(
      h, s0, cntp, ws0, _update_body,
      out_specs=[
          pl.BlockSpec((bn, H), lambda i: (i, 0)),
          pl.BlockSpec((2, bn, HH), lambda i: (0, i, 0)),
      ],
      out_shape=[
          jax.ShapeDtypeStruct((N_NODES, H), F32),
          jax.ShapeDtypeStruct((2, N_NODES, HH), F32),
      ])

  ha_flat = ha1.reshape(2 * N_NODES, HH)
  s1 = _sc_mp(H, srcs, dsts, ha_flat, eb3, zeros_f)
  ws1 = [mp1_mW2.T, r2(mp1_mb2), mp1_uW1[:, :H].T, mp1_uW1[:, H:].T,
         r2(mp1_ub1), mp1_uW2.T, r2(mp1_ub2),
         dec_W1.T, r2(dec_b1), dec_W2.T, r2(dec_b2)]
  out = _update(
      h, s1, cntp, ws1, _update_dec_body,
      out_specs=[pl.BlockSpec((bn, 1), lambda i: (i, 0))],
      out_shape=[jax.ShapeDtypeStruct((N_NODES, 1), F32)])
  return out[0].reshape(N_NODES)
